# baseline (speedup 1.0000x reference)
import math

import jax
import jax.numpy as jnp
from jax import lax
from jax.experimental import pallas as pl
from jax.experimental.pallas import tpu as pltpu
from jax.experimental.pallas import tpu_sc as plsc

N = 10000
M = 10000
H = 32
IN = 128
OUT = 128
K = 27
KS = 3
P = 3
RADIUS = 0.5
KP_EXTENT = 2.0 * RADIUS / (KS - 1) / math.sqrt(P)

BQ = 256
NP_ = 10240
NBLK = NP_ // BQ
B = NP_ * H

NW = 32
NC = 2
BPW = B // NW
CHUNK = 128
ROUNDS = BPW // CHUNK


def _sc_pos_kernel(sx_hbm, sy_hbm, sz_hbm, idx_hbm,
                   osx_hbm, osy_hbm, osz_hbm,
                   idxf, sgx, sgy, sgz, wsem):
    wid = lax.axis_index("s") * NC + lax.axis_index("c")
    base = wid * BPW
    pltpu.sync_copy(idx_hbm.at[pl.ds(base, BPW)], idxf)
    cols = ((sx_hbm, sgx), (sy_hbm, sgy), (sz_hbm, sgz))
    copies = [pltpu.async_copy(col_hbm.at[idxf], sg, wsem)
              for col_hbm, sg in cols]
    for c in copies:
        c.wait()
    for (_, sg), out in zip(cols, (osx_hbm, osy_hbm, osz_hbm)):
        pltpu.sync_copy(sg, out.at[pl.ds(base, BPW)])


def _sc_xgather_kernel(x_hbm, idx_hbm, outx_hbm,
                       idxf, buf0, buf1, buf2, buf3,
                       gsem0, gsem1, gsem2, gsem3,
                       osem0, osem1, osem2, osem3):
    wid = lax.axis_index("s") * NC + lax.axis_index("c")
    base = wid * BPW
    pltpu.sync_copy(idx_hbm.at[pl.ds(base, BPW)], idxf)
    bufs = (buf0, buf1, buf2, buf3)
    gsems = (gsem0, gsem1, gsem2, gsem3)
    osems = (osem0, osem1, osem2, osem3)

    def g_copy(r, b):
        return pltpu.make_async_copy(
            x_hbm.at[idxf.at[pl.ds(r * CHUNK, CHUNK)]], bufs[b], gsems[b])

    def o_copy(r, b):
        return pltpu.make_async_copy(
            bufs[b], outx_hbm.at[pl.ds(base + r * CHUNK, CHUNK)], osems[b])

    g_copy(0, 0).start()
    g_copy(1, 1).start()

    def round_quad(rr, _):
        for b in range(4):
            r = rr * 4 + b
            g_copy(r, b).wait()
            o_copy(r, b).start()
            bn = (b + 2) % 4

            @pl.when(r + 2 < ROUNDS)
            def _():
                pl.when(r >= 2)(lambda: o_copy(r - 2, bn).wait())
                g_copy(r + 2, bn).start()
        return 0

    lax.fori_loop(0, ROUNDS // 4, round_quad, 0)
    for last in range(ROUNDS - 4, ROUNDS):
        o_copy(last, last % 4).wait()


def _tc_kernel(xg_ref, sg_ref, q_ref, kpt_ref, w_ref, out_ref):
    s = sg_ref[...]
    q = q_ref[...]
    rel = (s.reshape(BQ, H, 8) - q[:, None, :]).reshape(BQ * H, 8)
    rel2 = jnp.sum(rel * rel, axis=1, keepdims=True) - 1.0
    mm = jnp.dot(rel, kpt_ref[...], preferred_element_type=jnp.float32,
                 precision=lax.Precision.HIGHEST)
    sq_d = jnp.maximum(rel2 + mm, 0.0)
    w = jnp.maximum(1.0 - jnp.sqrt(sq_d) * (1.0 / KP_EXTENT), 0.0)
    w3 = w.reshape(BQ, H, 32)
    x3 = xg_ref[...].reshape(BQ, H, IN)
    g = lax.dot_general(w3, x3, (((1,), (1,)), ((0,), (0,))),
                        preferred_element_type=jnp.float32)
    acc = jnp.zeros((BQ, OUT), dtype=jnp.float32)
    for k in range(K):
        acc += jnp.dot(g[:, k, :], w_ref[k],
                       preferred_element_type=jnp.float32)
    out_ref[...] = acc


def kernel(q_pts, s_pts, neighb_inds, x, weights, kernel_points):
    idx = neighb_inds.astype(jnp.int32).reshape(-1)
    idx = jnp.concatenate(
        [idx, jnp.zeros((B - N * H,), dtype=jnp.int32)])

    sx, sy, sz = s_pts[:, 0], s_pts[:, 1], s_pts[:, 2]

    q8 = jnp.concatenate(
        [q_pts, jnp.zeros((N, 5), jnp.float32)], axis=1)
    q8 = jnp.concatenate(
        [q8, jnp.zeros((NP_ - N, 8), jnp.float32)], axis=0)

    kp2 = jnp.sum(kernel_points * kernel_points, axis=1)
    kpt = jnp.zeros((8, 32), jnp.float32)
    kpt = kpt.at[:3, :K].set(-2.0 * kernel_points.T)
    kpt = kpt.at[3, :K].set(kp2)
    kpt = kpt.at[3, K:].set(1e9)

    mesh = plsc.VectorSubcoreMesh(core_axis_name="c", subcore_axis_name="s")

    sgx, sgy, sgz = pl.kernel(
        _sc_pos_kernel,
        out_type=(
            jax.ShapeDtypeStruct((B,), jnp.float32),
            jax.ShapeDtypeStruct((B,), jnp.float32),
            jax.ShapeDtypeStruct((B,), jnp.float32),
        ),
        mesh=mesh,
        scratch_types=[
            pltpu.VMEM((BPW,), jnp.int32),
            pltpu.VMEM((BPW,), jnp.float32),
            pltpu.VMEM((BPW,), jnp.float32),
            pltpu.VMEM((BPW,), jnp.float32),
            pltpu.SemaphoreType.DMA,
        ],
    )(sx, sy, sz, idx)

    xg = pl.kernel(
        _sc_xgather_kernel,
        out_type=jax.ShapeDtypeStruct((B, IN), jnp.float32),
        mesh=mesh,
        scratch_types=[
            pltpu.VMEM((BPW,), jnp.int32),
            pltpu.VMEM((CHUNK, IN), jnp.float32),
            pltpu.VMEM((CHUNK, IN), jnp.float32),
            pltpu.VMEM((CHUNK, IN), jnp.float32),
            pltpu.VMEM((CHUNK, IN), jnp.float32),
        ] + [pltpu.SemaphoreType.DMA] * 8,
    )(x, idx)

    s8 = jnp.stack(
        [sgx, sgy, sgz, jnp.ones((B,), jnp.float32)] +
        [jnp.zeros((B,), jnp.float32)] * 4, axis=1)

    fx = pl.pallas_call(
        _tc_kernel,
        grid=(NBLK,),
        in_specs=[
            pl.BlockSpec((BQ * H, IN), lambda i: (i, 0)),
            pl.BlockSpec((BQ * H, 8), lambda i: (i, 0)),
            pl.BlockSpec((BQ, 8), lambda i: (i, 0)),
            pl.BlockSpec((8, 32), lambda i: (0, 0)),
            pl.BlockSpec((K, IN, OUT), lambda i: (0, 0, 0)),
        ],
        out_specs=pl.BlockSpec((BQ, OUT), lambda i: (i, 0)),
        out_shape=jax.ShapeDtypeStruct((NP_, OUT), jnp.float32),
    )(xg, s8, q8, kpt, weights)

    return fx[:N]


# speedup vs baseline: 1.0817x; 1.0817x over previous
import math

import jax
import jax.numpy as jnp
from jax import lax
from jax.experimental import pallas as pl
from jax.experimental.pallas import tpu as pltpu
from jax.experimental.pallas import tpu_sc as plsc

N = 10000
M = 10000
H = 32
IN = 128
OUT = 128
K = 27
KS = 3
P = 3
RADIUS = 0.5
KP_EXTENT = 2.0 * RADIUS / (KS - 1) / math.sqrt(P)

BQ = 256
NP_ = 10240
NBLK = NP_ // BQ
B = NP_ * H

NW = 32
NC = 2
BPW = B // NW
CHUNK = 128
ROUNDS = BPW // CHUNK


def _sc_pos_kernel(sx_hbm, sy_hbm, sz_hbm, idx_hbm,
                   osx_hbm, osy_hbm, osz_hbm,
                   idxf, sgx, sgy, sgz, wsem):
    wid = lax.axis_index("s") * NC + lax.axis_index("c")
    base = wid * BPW
    pltpu.sync_copy(idx_hbm.at[pl.ds(base, BPW)], idxf)
    cols = ((sx_hbm, sgx), (sy_hbm, sgy), (sz_hbm, sgz))

    def w_copy(row, col_hbm, sg):
        return pltpu.make_async_copy(
            col_hbm.at[idxf.at[pl.ds(row * 128, 128)]],
            sg.at[pl.ds(row * 128, 128)], wsem)

    def fire(row, _):
        for col_hbm, sg in cols:
            w_copy(row, col_hbm, sg).start()
        return 0

    lax.fori_loop(0, BPW // 128, fire, 0)

    def drain(row, _):
        for col_hbm, sg in cols:
            w_copy(row, col_hbm, sg).wait()
        return 0

    lax.fori_loop(0, BPW // 128, drain, 0)
    for (_, sg), out in zip(cols, (osx_hbm, osy_hbm, osz_hbm)):
        pltpu.sync_copy(sg, out.at[pl.ds(base, BPW)])


def _sc_xgather_kernel(x_hbm, idx_hbm, outx_hbm,
                       idxf, buf0, buf1, buf2, buf3,
                       gsem0, gsem1, gsem2, gsem3,
                       osem0, osem1, osem2, osem3):
    wid = lax.axis_index("s") * NC + lax.axis_index("c")
    base = wid * BPW
    pltpu.sync_copy(idx_hbm.at[pl.ds(base, BPW)], idxf)
    bufs = (buf0, buf1, buf2, buf3)
    gsems = (gsem0, gsem1, gsem2, gsem3)
    osems = (osem0, osem1, osem2, osem3)

    def g_copy(r, b):
        return pltpu.make_async_copy(
            x_hbm.at[idxf.at[pl.ds(r * CHUNK, CHUNK)]], bufs[b], gsems[b])

    def o_copy(r, b):
        return pltpu.make_async_copy(
            bufs[b], outx_hbm.at[pl.ds(base + r * CHUNK, CHUNK)], osems[b])

    g_copy(0, 0).start()
    g_copy(1, 1).start()

    def round_quad(rr, _):
        for b in range(4):
            r = rr * 4 + b
            g_copy(r, b).wait()
            o_copy(r, b).start()
            bn = (b + 2) % 4

            @pl.when(r + 2 < ROUNDS)
            def _():
                pl.when(r >= 2)(lambda: o_copy(r - 2, bn).wait())
                g_copy(r + 2, bn).start()
        return 0

    lax.fori_loop(0, ROUNDS // 4, round_quad, 0)
    for last in range(ROUNDS - 4, ROUNDS):
        o_copy(last, last % 4).wait()


def _tc_kernel(xg_ref, sg_ref, q_ref, kpt_ref, w_ref, out_ref):
    s = sg_ref[...]
    q = q_ref[...]
    rel = (s.reshape(BQ, H, 8) - q[:, None, :]).reshape(BQ * H, 8)
    rel2 = jnp.sum(rel * rel, axis=1, keepdims=True) - 1.0
    mm = jnp.dot(rel, kpt_ref[...], preferred_element_type=jnp.float32,
                 precision=lax.Precision.HIGHEST)
    sq_d = jnp.maximum(rel2 + mm, 0.0)
    w = jnp.maximum(1.0 - jnp.sqrt(sq_d) * (1.0 / KP_EXTENT), 0.0)
    w3 = w.reshape(BQ, H, 32)
    x3 = xg_ref[...].reshape(BQ, H, IN)
    g = lax.dot_general(w3, x3, (((1,), (1,)), ((0,), (0,))),
                        preferred_element_type=jnp.float32)
    acc = jnp.zeros((BQ, OUT), dtype=jnp.float32)
    for k in range(K):
        acc += jnp.dot(g[:, k, :], w_ref[k],
                       preferred_element_type=jnp.float32)
    out_ref[...] = acc


def kernel(q_pts, s_pts, neighb_inds, x, weights, kernel_points):
    idx = neighb_inds.astype(jnp.int32).reshape(-1)
    idx = jnp.concatenate(
        [idx, jnp.zeros((B - N * H,), dtype=jnp.int32)])

    sx, sy, sz = s_pts[:, 0], s_pts[:, 1], s_pts[:, 2]

    q8 = jnp.concatenate(
        [q_pts, jnp.zeros((N, 5), jnp.float32)], axis=1)
    q8 = jnp.concatenate(
        [q8, jnp.zeros((NP_ - N, 8), jnp.float32)], axis=0)

    kp2 = jnp.sum(kernel_points * kernel_points, axis=1)
    kpt = jnp.zeros((8, 32), jnp.float32)
    kpt = kpt.at[:3, :K].set(-2.0 * kernel_points.T)
    kpt = kpt.at[3, :K].set(kp2)
    kpt = kpt.at[3, K:].set(1e9)

    mesh = plsc.VectorSubcoreMesh(core_axis_name="c", subcore_axis_name="s")

    sgx, sgy, sgz = pl.kernel(
        _sc_pos_kernel,
        out_type=(
            jax.ShapeDtypeStruct((B,), jnp.float32),
            jax.ShapeDtypeStruct((B,), jnp.float32),
            jax.ShapeDtypeStruct((B,), jnp.float32),
        ),
        mesh=mesh,
        scratch_types=[
            pltpu.VMEM((BPW,), jnp.int32),
            pltpu.VMEM((BPW,), jnp.float32),
            pltpu.VMEM((BPW,), jnp.float32),
            pltpu.VMEM((BPW,), jnp.float32),
            pltpu.SemaphoreType.DMA,
        ],
    )(sx, sy, sz, idx)

    xg = pl.kernel(
        _sc_xgather_kernel,
        out_type=jax.ShapeDtypeStruct((B, IN), jnp.float32),
        mesh=mesh,
        scratch_types=[
            pltpu.VMEM((BPW,), jnp.int32),
            pltpu.VMEM((CHUNK, IN), jnp.float32),
            pltpu.VMEM((CHUNK, IN), jnp.float32),
            pltpu.VMEM((CHUNK, IN), jnp.float32),
            pltpu.VMEM((CHUNK, IN), jnp.float32),
        ] + [pltpu.SemaphoreType.DMA] * 8,
    )(x, idx)

    s8 = jnp.stack(
        [sgx, sgy, sgz, jnp.ones((B,), jnp.float32)] +
        [jnp.zeros((B,), jnp.float32)] * 4, axis=1)

    fx = pl.pallas_call(
        _tc_kernel,
        grid=(NBLK,),
        in_specs=[
            pl.BlockSpec((BQ * H, IN), lambda i: (i, 0)),
            pl.BlockSpec((BQ * H, 8), lambda i: (i, 0)),
            pl.BlockSpec((BQ, 8), lambda i: (i, 0)),
            pl.BlockSpec((8, 32), lambda i: (0, 0)),
            pl.BlockSpec((K, IN, OUT), lambda i: (0, 0, 0)),
        ],
        out_specs=pl.BlockSpec((BQ, OUT), lambda i: (i, 0)),
        out_shape=jax.ShapeDtypeStruct((NP_, OUT), jnp.float32),
    )(xg, s8, q8, kpt, weights)

    return fx[:N]
